# baseline (speedup 1.0000x reference)
import functools

import jax
import jax.numpy as jnp
from jax import lax
from jax.experimental import pallas as pl
from jax.experimental.pallas import tpu as pltpu
from jax.experimental.pallas import tpu_sc as plsc

_B, _F, _T = 64, 128, 4096
_NW = 32
_BPW = _B // _NW
_NTM = 10
_GR = 4
_NG = _F // _GR
_NBUF = 6
_LOOK = 3
_NCH = _T // 16

_mesh = plsc.VectorSubcoreMesh(core_axis_name="c", subcore_axis_name="s")


@functools.partial(
    pl.kernel,
    out_type=jax.ShapeDtypeStruct((_B, _F, _T), jnp.float32),
    mesh=_mesh,
    scratch_types=[
        pltpu.VMEM((_NBUF, _GR, _T), jnp.float32),
        pltpu.VMEM((_BPW, _T), jnp.float32),
        pltpu.VMEM((_B, 16), jnp.int32),
        pltpu.VMEM((4, 16), jnp.int32),
        pltpu.SemaphoreType.DMA((_NBUF,)),
        pltpu.SemaphoreType.DMA((_NBUF,)),
    ],
)
def _sc_run(x_hbm, xlb_hbm, prm_hbm, out_hbm,
            gbuf, keeprow, xl_v, prm_v, sem_in, sem_out):
    wid = lax.axis_index("s") * 2 + lax.axis_index("c")

    zv = jnp.zeros((16,), jnp.float32)
    ones = jnp.ones((16,), jnp.float32)

    pltpu.sync_copy(xlb_hbm, xl_v)
    pltpu.sync_copy(prm_hbm, prm_v)

    ts_v = prm_v[0]
    tw_v = prm_v[1]
    fs_v = prm_v[2]
    fe_v = fs_v + prm_v[3]
    fs0, fe0 = fs_v[0], fe_v[0]
    fs1, fe1 = fs_v[1], fe_v[1]

    def _is_freq(f):
        return ((f >= fs0) & (f < fe0)) | ((f >= fs1) & (f < fe1))

    def _full_freq(g):
        full = _is_freq(g * _GR)
        for r in range(1, _GR):
            full = full & _is_freq(g * _GR + r)
        return full

    def _any_freq(g):
        anyf = _is_freq(g * _GR)
        for r in range(1, _GR):
            anyf = anyf | _is_freq(g * _GR + r)
        return anyf

    def _g_in(b, g, slot):
        off = pl.multiple_of(g * _GR, _GR)
        pltpu.async_copy(x_hbm.at[b, pl.ds(off, _GR), :], gbuf.at[slot],
                         sem_in.at[slot])

    def _g_in_wait(b, slot):
        pltpu.make_async_copy(x_hbm.at[b, pl.ds(0, _GR), :], gbuf.at[slot],
                              sem_in.at[slot]).wait()

    def _g_out(b, g, slot):
        off = pl.multiple_of(g * _GR, _GR)
        pltpu.async_copy(gbuf.at[slot], out_hbm.at[b, pl.ds(off, _GR), :],
                         sem_out.at[slot])

    def _g_out_wait(b, slot):
        pltpu.make_async_copy(gbuf.at[slot], out_hbm.at[b, pl.ds(0, _GR), :],
                              sem_out.at[slot]).wait()

    clo_all, chi_all = [], []
    for bi in range(_BPW):
        b = wid * _BPW + bi
        xlv = xl_v[b]
        s_vec = jnp.minimum(ts_v, xlv)
        e_vec = jnp.minimum(ts_v + tw_v, xlv)
        c0_vec = (s_vec + 15) >> 4
        c1_vec = e_vec >> 4
        clo_all.append([(s_vec >> 4)[i] for i in range(_NTM)])
        chi_all.append([((e_vec + 15) >> 4)[i] for i in range(_NTM)])

        def _init(i, carry):
            keeprow[bi, pl.ds(i * 16, 16)] = ones
            return carry

        lax.fori_loop(0, _NCH, _init, 0)

        for i in range(_NTM):
            s_i, e_i = s_vec[i], e_vec[i]

            @pl.when(s_i < e_i)
            def _():
                def _zero(c, carry):
                    keeprow[bi, pl.ds(c * 16, 16)] = zv
                    return carry

                lax.fori_loop(c0_vec[i], c1_vec[i], _zero, 0)

                def _edge(ec):
                    tvec = lax.iota(jnp.int32, 16) + ec * 16
                    m = (tvec >= s_i) & (tvec < e_i)
                    cur = keeprow[bi, pl.ds(ec * 16, 16)]
                    keeprow[bi, pl.ds(ec * 16, 16)] = jnp.where(m, 0.0, cur)

                fix_l = (s_i & 15) != 0
                fix_r = ((e_i & 15) != 0) & (
                    jnp.logical_not(fix_l) | ((e_i >> 4) != (s_i >> 4)))

                @pl.when(fix_l)
                def _():
                    _edge(s_i >> 4)

                @pl.when(fix_r)
                def _():
                    _edge(e_i >> 4)

    _GG = _BPW * _NG

    def _bat(gg):
        return gg // _NG

    def _grp(gg):
        return gg % _NG

    for g0 in range(_LOOK):
        @pl.when(jnp.logical_not(_full_freq(_grp(g0))))
        def _():
            _g_in(wid * _BPW + _bat(g0), _grp(g0), g0 % _NBUF)

    def _gstep(gg, carry):
        slot = gg % _NBUF
        h = gg + _LOOK

        @pl.when(h < _GG)
        def _():
            hslot = h % _NBUF

            @pl.when(h >= _NBUF)
            def _():
                _g_out_wait(wid * _BPW, hslot)

            @pl.when(jnp.logical_not(_full_freq(_grp(h))))
            def _():
                _g_in(wid * _BPW + _bat(h), _grp(h), hslot)

        g = _grp(gg)
        kb = _bat(gg)
        b = wid * _BPW + kb
        full = _full_freq(g)

        @pl.when(full)
        def _():
            def _zg(c, carry2):
                for r in range(_GR):
                    gbuf[slot, r, pl.ds(c * 16, 16)] = zv
                return carry2

            lax.fori_loop(0, _NCH, _zg, 0)

        @pl.when(jnp.logical_not(full))
        def _():
            _g_in_wait(b, slot)

            @pl.when(_any_freq(g))
            def _():
                for r in range(_GR):
                    @pl.when(_is_freq(g * _GR + r))
                    def _():
                        def _zr(c, carry2):
                            gbuf[slot, r, pl.ds(c * 16, 16)] = zv
                            return carry2

                        lax.fori_loop(0, _NCH, _zr, 0)

            for i in range(_NTM):
                cl = jnp.where(kb == 0, clo_all[0][i], clo_all[1][i])
                ch = jnp.where(kb == 0, chi_all[0][i], chi_all[1][i])

                def _mul(c, carry2):
                    k = keeprow[kb, pl.ds(c * 16, 16)]
                    for r in range(_GR):
                        v = gbuf[slot, r, pl.ds(c * 16, 16)]
                        gbuf[slot, r, pl.ds(c * 16, 16)] = v * k
                    return carry2

                lax.fori_loop(cl, ch, _mul, 0)

        _g_out(b, g, slot)
        return carry

    lax.fori_loop(0, _GG, _gstep, 0)

    def _gdrain(gg, carry):
        _g_out_wait(wid * _BPW, gg % _NBUF)
        return carry

    lax.fori_loop(_GG - _NBUF, _GG, _gdrain, 0)


def kernel(x, x_len, freq_starts, freq_widths, time_starts, time_widths):
    xl = x_len.astype(jnp.int32)
    xlb = jnp.tile(xl[:, None], (1, 16))
    pad6 = jnp.zeros((6,), jnp.int32)
    pad14 = jnp.zeros((14,), jnp.int32)
    prm = jnp.stack([
        jnp.concatenate([time_starts.astype(jnp.int32), pad6]),
        jnp.concatenate([time_widths.astype(jnp.int32), pad6]),
        jnp.concatenate([freq_starts.astype(jnp.int32), pad14]),
        jnp.concatenate([freq_widths.astype(jnp.int32), pad14]),
    ])
    return _sc_run(x, xlb, prm)
